# baseline (speedup 1.0000x reference)
import functools

import jax
import jax.numpy as jnp
from jax.experimental import pallas as pl
from jax.experimental.pallas import tpu as pltpu

B, L, D, D_OUT = 16, 2048, 1024, 1024
ROWS = 2


def _body(x_ref, mask_ref, w1_ref, v_ref, wr_ref, br_ref, out_ref, pool_ref):
    g = pl.program_id(0)

    for r in range(ROWS):
        x = x_ref[r]
        xb = x.astype(jnp.bfloat16)
        h = jnp.tanh(
            jax.lax.dot_general(xb, w1_ref[...], (((1,), (0,)), ((), ())),
                                preferred_element_type=jnp.float32))
        scores = jnp.sum(h * v_ref[...], axis=1, keepdims=True)
        scores = jnp.where(mask_ref[r] > 0, scores, jnp.float32(-1e9))
        m = jnp.max(scores)
        p = jnp.exp(scores - m)
        pw = p / jnp.sum(p)
        pooled = jax.lax.dot_general(pw, x, (((0,), (0,)), ((), ())),
                                     preferred_element_type=jnp.float32)
        pool_ref[pl.ds(g * ROWS + r, 1), :] = pooled

    @pl.when(g == B // ROWS - 1)
    def _finish():
        out_ref[...] = jax.lax.dot_general(
            pool_ref[...], wr_ref[...], (((1,), (0,)), ((), ())),
            preferred_element_type=jnp.float32) + br_ref[...]


@functools.partial(jax.jit, static_argnames=())
def kernel(word_hidden, word_mask, W1, v, Wr, br):
    maskf = word_mask.astype(jnp.float32).reshape(B, L, 1)
    w1_bf = W1.astype(jnp.bfloat16)
    v2 = v.reshape(1, D)
    br2 = br.reshape(1, D_OUT)
    out = pl.pallas_call(
        _body,
        grid=(B // ROWS,),
        in_specs=[
            pl.BlockSpec((ROWS, L, D), lambda b: (b, 0, 0)),
            pl.BlockSpec((ROWS, L, 1), lambda b: (b, 0, 0)),
            pl.BlockSpec((D, D), lambda b: (0, 0)),
            pl.BlockSpec((1, D), lambda b: (0, 0)),
            pl.BlockSpec((D, D_OUT), lambda b: (0, 0)),
            pl.BlockSpec((1, D_OUT), lambda b: (0, 0)),
        ],
        out_specs=pl.BlockSpec((B, D_OUT), lambda b: (0, 0)),
        out_shape=jax.ShapeDtypeStruct((B, D_OUT), jnp.float32),
        scratch_shapes=[
            pltpu.VMEM((B, D), jnp.float32),
        ],
        compiler_params=pltpu.CompilerParams(
            dimension_semantics=("arbitrary",)),
    )(word_hidden, maskf, w1_bf, v2, Wr, br2)
    return out


# speedup vs baseline: 1.0348x; 1.0348x over previous
import functools

import jax
import jax.numpy as jnp
from jax.experimental import pallas as pl
from jax.experimental.pallas import tpu as pltpu

B, L, D, D_OUT = 16, 2048, 1024, 1024
L2 = L // 2


def _half(x_ref, w1_ref, v_ref, mask, lo):
    xb = x_ref[0, 0].astype(jnp.bfloat16)
    h = jnp.tanh(
        jax.lax.dot_general(xb, w1_ref[...], (((1,), (0,)), ((), ())),
                            preferred_element_type=jnp.float32))
    s = jnp.sum(h * v_ref[...], axis=1, keepdims=True)
    s = jnp.where(mask[lo:lo + L2] > 0, s, jnp.float32(-1e9))
    return xb, s


def _body(x0_ref, x1_ref, mask_ref, w1_ref, v_ref, wr_ref, br_ref, out_ref,
          pool_ref):
    b = pl.program_id(0)
    mask = mask_ref[0]

    xb0, s0 = _half(x0_ref, w1_ref, v_ref, mask, 0)
    xb1, s1 = _half(x1_ref, w1_ref, v_ref, mask, L2)

    scores = jnp.concatenate([s0, s1], axis=0)
    m = jnp.max(scores)
    p = jnp.exp(scores - m)
    pw = (p / jnp.sum(p)).astype(jnp.bfloat16)
    pooled = (
        jax.lax.dot_general(pw[:L2], xb0, (((0,), (0,)), ((), ())),
                            preferred_element_type=jnp.float32)
        + jax.lax.dot_general(pw[L2:], xb1, (((0,), (0,)), ((), ())),
                              preferred_element_type=jnp.float32))
    pool_ref[pl.ds(b, 1), :] = pooled

    @pl.when(b == B - 1)
    def _finish():
        out_ref[...] = jax.lax.dot_general(
            pool_ref[...], wr_ref[...], (((1,), (0,)), ((), ())),
            preferred_element_type=jnp.float32) + br_ref[...]


@functools.partial(jax.jit, static_argnames=())
def kernel(word_hidden, word_mask, W1, v, Wr, br):
    maskf = word_mask.astype(jnp.float32).reshape(B, L, 1)
    xs = word_hidden.reshape(B, 2, L2, D)
    w1_bf = W1.astype(jnp.bfloat16)
    v2 = v.reshape(1, D)
    br2 = br.reshape(1, D_OUT)
    out = pl.pallas_call(
        _body,
        grid=(B,),
        in_specs=[
            pl.BlockSpec((1, 1, L2, D), lambda b: (b, 0, 0, 0)),
            pl.BlockSpec((1, 1, L2, D), lambda b: (b, 1, 0, 0)),
            pl.BlockSpec((1, L, 1), lambda b: (b, 0, 0)),
            pl.BlockSpec((D, D), lambda b: (0, 0)),
            pl.BlockSpec((1, D), lambda b: (0, 0)),
            pl.BlockSpec((D, D_OUT), lambda b: (0, 0)),
            pl.BlockSpec((1, D_OUT), lambda b: (0, 0)),
        ],
        out_specs=pl.BlockSpec((B, D_OUT), lambda b: (0, 0)),
        out_shape=jax.ShapeDtypeStruct((B, D_OUT), jnp.float32),
        scratch_shapes=[
            pltpu.VMEM((B, D), jnp.float32),
        ],
        compiler_params=pltpu.CompilerParams(
            dimension_semantics=("arbitrary",)),
    )(xs, xs, maskf, w1_bf, v2, Wr, br2)
    return out
